# Initial kernel scaffold; baseline (speedup 1.0000x reference)
import jax
import jax.numpy as jnp
from jax.experimental import pallas as pl


def kernel(rainfall, inflow, edge_index, lstm_W_ih, lstm_W_hh, lstm_b_ih, lstm_b_hh, fc_W, fc_b, c1_W, c1_asrc, c1_adst, c1_b, c2_W, c2_asrc, c2_adst, c2_b, cell_W_ih, cell_W_hh, cell_b_ih, cell_b_hh, ln_g, ln_b, lin_W, lin_b):
    raise NotImplementedError("write your pallas kernel here")


# speedup vs baseline: 1.0004x; 1.0004x over previous
import jax
import jax.numpy as jnp
from jax.experimental import pallas as pl
from jax.experimental.pallas import tpu as pltpu

NUM_NODES = 10000
B = 4
T = 12
H_RR = 64
H_RT = 256
HEADS = 8


def _runoff_kernel(rain_ref, w_ih_ref, w_hh_ref, b_ih_ref, b_hh_ref,
                   fcw_ref, fcb_ref, out_ref):
    rain = rain_ref[...]
    w_ih = w_ih_ref[...]
    w_hh = w_hh_ref[...]
    b_all = b_ih_ref[...] + b_hh_ref[...]
    h = jnp.zeros((B, H_RR), dtype=jnp.float32)
    c = jnp.zeros((B, H_RR), dtype=jnp.float32)
    outs = []
    for t in range(T):
        g = rain[:, t:t + 1] * w_ih[:, 0][None, :] + b_all
        g = g + jnp.dot(h, w_hh.T, preferred_element_type=jnp.float32)
        ig, fg, gg, og = jnp.split(g, 4, axis=1)
        c = jax.nn.sigmoid(fg) * c + jax.nn.sigmoid(ig) * jnp.tanh(gg)
        h = jax.nn.sigmoid(og) * jnp.tanh(c)
        outs.append(h)
    lstm_out = jnp.concatenate(outs, axis=0)
    r = jnp.dot(lstm_out, fcw_ref[...].T, preferred_element_type=jnp.float32)
    r = r + fcb_ref[...]
    out_ref[...] = jax.nn.leaky_relu(r, 0.01)


def _runoff(rainfall, lstm_W_ih, lstm_W_hh, lstm_b_ih, lstm_b_hh, fc_W, fc_b):
    rain = rainfall[:, :, 0]
    out = pl.pallas_call(
        _runoff_kernel,
        out_shape=jax.ShapeDtypeStruct((T * B, NUM_NODES), jnp.float32),
    )(rain, lstm_W_ih, lstm_W_hh, lstm_b_ih[None, :], lstm_b_hh[None, :],
      fc_W, fc_b[None, :])
    return out.reshape(T, B, NUM_NODES).transpose(1, 0, 2)


def _gat_layer(x, src, dst, W, asrc, adst, b, heads, C, concat, N):
    h = (x @ W.T).reshape(-1, heads, C)
    a_s = (h * asrc).sum(-1)
    a_d = (h * adst).sum(-1)
    alpha = a_s[src] + a_d[dst]
    alpha = jnp.where(alpha > 0, alpha, 0.2 * alpha)
    amax = jax.ops.segment_max(alpha, dst, num_segments=N)
    amax = jax.lax.stop_gradient(jnp.where(jnp.isfinite(amax), amax, 0.0))
    ex = jnp.exp(alpha - amax[dst])
    den = jax.ops.segment_sum(ex, dst, num_segments=N)
    att = ex / (den[dst] + 1e-16)
    out = jax.ops.segment_sum(h[src] * att[..., None], dst, num_segments=N)
    if concat:
        out = out.reshape(N, heads * C)
    else:
        out = out.mean(axis=1)
    return out + b, att


def kernel(rainfall, inflow, edge_index, lstm_W_ih, lstm_W_hh, lstm_b_ih,
           lstm_b_hh, fc_W, fc_b, c1_W, c1_asrc, c1_adst, c1_b, c2_W,
           c2_asrc, c2_adst, c2_b, cell_W_ih, cell_W_hh, cell_b_ih,
           cell_b_hh, ln_g, ln_b, lin_W, lin_b):
    batch = rainfall.shape[0]
    seq_len = rainfall.shape[1]
    N = NUM_NODES
    offs = (jnp.arange(batch, dtype=edge_index.dtype) * N)
    ei = (edge_index[:, None, :] + offs[None, :, None]).reshape(2, -1)
    BN = batch * N
    loops = jnp.arange(BN, dtype=ei.dtype)
    src = jnp.concatenate([ei[0], loops])
    dst = jnp.concatenate([ei[1], loops])

    runoff = _runoff(rainfall, lstm_W_ih, lstm_W_hh, lstm_b_ih, lstm_b_hh,
                     fc_W, fc_b)

    hn = jnp.zeros((batch, H_RT), dtype=jnp.float32)
    cn = jnp.zeros((batch, H_RT), dtype=jnp.float32)
    xn = jnp.zeros((batch, N, 2), dtype=jnp.float32)
    preds, lats, atts = [], [], []
    for t in range(seq_len):
        cr = runoff[:, t, :].at[:, 753].add(inflow[:, t, 0])
        lat = cr[:, :, None]
        x = jnp.concatenate([xn, lat], axis=-1).reshape(BN, 3)
        x, att = _gat_layer(x, src, dst, c1_W, c1_asrc, c1_adst, c1_b,
                            HEADS, 3, True, BN)
        x = jax.nn.leaky_relu(x, 0.01)
        x, _ = _gat_layer(x, src, dst, c2_W, c2_asrc, c2_adst, c2_b,
                          HEADS, 2, False, BN)
        x = jax.nn.leaky_relu(x, 0.01)
        x = x.reshape(batch, -1)
        g = x @ cell_W_ih.T + cell_b_ih + hn @ cell_W_hh.T + cell_b_hh
        ig, fg, gg, og = jnp.split(g, 4, axis=1)
        cn = jax.nn.sigmoid(fg) * cn + jax.nn.sigmoid(ig) * jnp.tanh(gg)
        hn = jax.nn.sigmoid(og) * jnp.tanh(cn)
        mu = hn.mean(axis=-1, keepdims=True)
        var = ((hn - mu) ** 2).mean(axis=-1, keepdims=True)
        hn = (hn - mu) / jnp.sqrt(var + 1e-5) * ln_g + ln_b
        xn_flat = jax.nn.softplus(hn @ lin_W.T + lin_b)
        preds.append(xn_flat)
        lats.append(lat)
        atts.append(att)
        xn = xn_flat.reshape(batch, N, 2)
    prediction = jnp.stack(preds, axis=1)
    Lateral = jnp.stack(lats, axis=1)
    Attention = jnp.stack(atts, axis=1)
    return (prediction, Lateral, Attention)


# speedup vs baseline: 41.4313x; 41.4147x over previous
import functools

import jax
import jax.numpy as jnp
from jax import lax
from jax.experimental import pallas as pl
from jax.experimental.pallas import tpu as pltpu
from jax.experimental.pallas import tpu_sc as plsc

NUM_NODES = 10000
B = 4
T = 12
H_RR = 64
H_RT = 256
HEADS = 8

BN = B * NUM_NODES
NC = 2
NS = 16
BNPAD = 40960
CH = 1024
HPC = HEADS // NC


def _runoff_kernel(rain_ref, w_ih_ref, w_hh_ref, b_ih_ref, b_hh_ref,
                   fcw_ref, fcb_ref, out_ref):
    rain = rain_ref[...]
    w_ih = w_ih_ref[...]
    w_hh = w_hh_ref[...]
    b_all = b_ih_ref[...] + b_hh_ref[...]
    h = jnp.zeros((B, H_RR), dtype=jnp.float32)
    c = jnp.zeros((B, H_RR), dtype=jnp.float32)
    outs = []
    for t in range(T):
        g = rain[:, t:t + 1] * w_ih[:, 0][None, :] + b_all
        g = g + jnp.dot(h, w_hh.T, preferred_element_type=jnp.float32)
        ig, fg, gg, og = jnp.split(g, 4, axis=1)
        c = jax.nn.sigmoid(fg) * c + jax.nn.sigmoid(ig) * jnp.tanh(gg)
        h = jax.nn.sigmoid(og) * jnp.tanh(c)
        outs.append(h)
    lstm_out = jnp.concatenate(outs, axis=0)
    r = jnp.dot(lstm_out, fcw_ref[...].T, preferred_element_type=jnp.float32)
    r = r + fcb_ref[...]
    out_ref[...] = jax.nn.leaky_relu(r, 0.01)


def _runoff(rainfall, lstm_W_ih, lstm_W_hh, lstm_b_ih, lstm_b_hh, fc_W, fc_b):
    rain = rainfall[:, :, 0]
    out = pl.pallas_call(
        _runoff_kernel,
        out_shape=jax.ShapeDtypeStruct((T * B, NUM_NODES), jnp.float32),
    )(rain, lstm_W_ih, lstm_W_hh, lstm_b_ih[None, :], lstm_b_hh[None, :],
      fc_W, fc_b[None, :])
    return out.reshape(T, B, NUM_NODES).transpose(1, 0, 2)


_SC_STAGE = 4


def _gat_edge_body(FH, NCHUNK, ETOTP,
                   asrc_hbm, adst_hbm, h_hbm, z4_hbm, zF_hbm,
                   src_hbm, dst_hbm,
                   att_hbm, out_hbm, den_hbm,
                   den_sh, out_sh,
                   sidx_v, didx_v, didx2_v,
                   asrc_v, adst_v, ex_v, den_v, att_v, h_v, ctr_v,
                   sem):
    c = lax.axis_index("c")
    tid = lax.axis_index("s")
    E_T = NCHUNK * CH
    rows_per_tile = BNPAD // NS
    CF = FH // HPC
    coff = c * BNPAD
    iota = lax.iota(jnp.int32, 16)

    r0 = tid * rows_per_tile
    pltpu.sync_copy(z4_hbm, den_sh.at[pl.ds(r0, rows_per_tile)])
    pltpu.sync_copy(zF_hbm, out_sh.at[pl.ds(r0, rows_per_tile)])
    plsc.subcore_barrier()

    ebase_rows = tid * (E_T // 128)
    nsub = CH // 128

    def load_idx(ch):
        rrow = ebase_rows + ch * nsub
        pltpu.sync_copy(src_hbm.at[pl.ds(rrow, nsub)], sidx_v)
        pltpu.sync_copy(dst_hbm.at[pl.ds(rrow, nsub)], didx_v)
        for jr in range(nsub):
            for jc in range(128 // 16):
                sl = (jr, pl.ds(jc * 16, 16))
                sidx_v[sl] = sidx_v[sl] + coff
                didx2_v[sl] = didx_v[sl] + coff

    def gather_alpha_inputs():
        cps = []
        for j in range(nsub):
            cps.append(pltpu.async_copy(
                asrc_hbm.at[sidx_v.at[j]],
                asrc_v.at[pl.ds(j * 128, 128)], sem))
            cps.append(pltpu.async_copy(
                adst_hbm.at[didx2_v.at[j]],
                adst_v.at[pl.ds(j * 128, 128)], sem))
        for cp in cps:
            cp.wait()

    def compute_ex(k, _):
        j0 = k * 16 + iota
        row = j0 // 4
        col = j0 - row * 4
        a = (plsc.load_gather(asrc_v, [row, col])
             + plsc.load_gather(adst_v, [row, col]))
        a = jnp.where(a > 0, a, 0.2 * a)
        plsc.store_scatter(ex_v, [row, col], jnp.exp(a))
        return 0

    def pass1(ch, _):
        load_idx(ch)
        gather_alpha_inputs()
        lax.fori_loop(0, CH * 4 // 16, compute_ex, 0)
        for j in range(nsub):
            pltpu.sync_copy(ex_v.at[pl.ds(j * 128, 128)],
                            den_sh.at[didx_v.at[j]], add=True)
        return 0

    if _SC_STAGE >= 2:
        lax.fori_loop(0, NCHUNK, pass1, 0)
    plsc.subcore_barrier()
    pltpu.sync_copy(den_sh.at[pl.ds(r0, rows_per_tile)],
                    den_hbm.at[pl.ds(coff + r0, rows_per_tile)])
    plsc.subcore_barrier()

    def pass2(ch, _):
        load_idx(ch)
        gather_alpha_inputs()
        cps = []
        for j in range(nsub):
            if _SC_STAGE >= 4:
                cps.append(pltpu.async_copy(
                    den_hbm.at[didx2_v.at[j]],
                    den_v.at[pl.ds(j * 128, 128)], sem))
            cps.append(pltpu.async_copy(
                h_hbm.at[sidx_v.at[j]],
                h_v.at[pl.ds(j * 128, 128)], sem))
        lax.fori_loop(0, CH * 4 // 16, compute_ex, 0)
        for cp in cps:
            cp.wait()

        def compute_att(k, _):
            j0 = k * 16 + iota
            row = j0 // 4
            col = j0 - row * 4
            e = plsc.load_gather(ex_v, [row, col])
            if _SC_STAGE >= 4:
                d = plsc.load_gather(den_v, [row, col])
                e = e / (d + 1e-16)
            plsc.store_scatter(att_v, [row, col], e)
            return 0

        lax.fori_loop(0, CH * 4 // 16, compute_att, 0)

        def compute_ctr(k, _):
            j0 = k * 16 + iota
            row = j0 // FH
            col = j0 - row * FH
            af = j0 // CF
            arow = af // 4
            acol = af - arow * 4
            av = plsc.load_gather(att_v, [arow, acol])
            hv = plsc.load_gather(h_v, [row, col])
            plsc.store_scatter(ctr_v, [row, col], av * hv)
            return 0

        lax.fori_loop(0, CH * FH // 16, compute_ctr, 0)

        arow0 = c * ETOTP + tid * E_T + ch * CH
        pltpu.sync_copy(att_v, att_hbm.at[pl.ds(arow0, CH)])
        for j in range(nsub):
            pltpu.sync_copy(ctr_v.at[pl.ds(j * 128, 128)],
                            out_sh.at[didx_v.at[j]], add=True)
        return 0

    if _SC_STAGE >= 3:
        lax.fori_loop(0, NCHUNK, pass2, 0)
    plsc.subcore_barrier()

    pltpu.sync_copy(out_sh.at[pl.ds(r0, rows_per_tile)],
                    out_hbm.at[pl.ds(coff + r0, rows_per_tile)])


def _make_gat_edge(FH, NCHUNK, ETOTP):
    mesh = plsc.VectorSubcoreMesh(core_axis_name="c", subcore_axis_name="s",
                                  num_cores=NC, num_subcores=NS)
    return pl.kernel(
        functools.partial(_gat_edge_body, FH, NCHUNK, ETOTP),
        out_type=[jax.ShapeDtypeStruct((2 * ETOTP, 4), jnp.float32),
                  jax.ShapeDtypeStruct((2 * BNPAD, FH), jnp.float32),
                  jax.ShapeDtypeStruct((2 * BNPAD, 4), jnp.float32)],
        mesh=mesh,
        compiler_params=pltpu.CompilerParams(use_tc_tiling_on_sc=False,
                                             needs_layout_passes=False),
        scratch_types=[
            pltpu.VMEM_SHARED((BNPAD, 4), jnp.float32),
            pltpu.VMEM_SHARED((BNPAD, FH), jnp.float32),
            pltpu.VMEM((CH // 128, 128), jnp.int32),
            pltpu.VMEM((CH // 128, 128), jnp.int32),
            pltpu.VMEM((CH // 128, 128), jnp.int32),
            pltpu.VMEM((CH, 4), jnp.float32),
            pltpu.VMEM((CH, 4), jnp.float32),
            pltpu.VMEM((CH, 4), jnp.float32),
            pltpu.VMEM((CH, 4), jnp.float32),
            pltpu.VMEM((CH, 4), jnp.float32),
            pltpu.VMEM((CH, FH), jnp.float32),
            pltpu.VMEM((CH, FH), jnp.float32),
            pltpu.SemaphoreType.DMA,
        ],
    )


def _pack_heads(a):
    p = a.reshape(BN, NC, HPC).transpose(1, 0, 2)
    p = jnp.pad(p, ((0, 0), (0, BNPAD - BN), (0, 0)))
    return p.reshape(NC * BNPAD, HPC)


def _pack_h(h, C):
    p = h.reshape(BN, NC, HPC, C).transpose(1, 0, 2, 3)
    p = p.reshape(NC, BN, HPC * C)
    p = jnp.pad(p, ((0, 0), (0, BNPAD - BN), (0, 0)))
    return p.reshape(NC * BNPAD, HPC * C)


def _gat_layer_sc(edge_fn, x, srcp, dstp, z4, zF, W, asrc, adst, b, C,
                  concat, ETOT, ETOTP):
    heads = HEADS
    h = (x @ W.T).reshape(BN, heads, C)
    a_s = (h * asrc).sum(-1)
    a_d = (h * adst).sum(-1)
    att2, out2, _ = edge_fn(_pack_heads(a_s), _pack_heads(a_d), _pack_h(h, C),
                            z4, zF, srcp, dstp)
    att = att2.reshape(NC, ETOTP, HPC)[:, :ETOT]
    att = att.transpose(1, 0, 2).reshape(ETOT, heads)
    out = out2.reshape(NC, BNPAD, HPC, C)[:, :BN]
    out = out.transpose(1, 0, 2, 3).reshape(BN, heads, C)
    if concat:
        out = out.reshape(BN, heads * C)
    else:
        out = out.mean(axis=1)
    return out + b, att


def _gat_layer_xla(x, src, dst, W, asrc, adst, b, heads, C, concat, N):
    h = (x @ W.T).reshape(-1, heads, C)
    a_s = (h * asrc).sum(-1)
    a_d = (h * adst).sum(-1)
    alpha = a_s[src] + a_d[dst]
    alpha = jnp.where(alpha > 0, alpha, 0.2 * alpha)
    ex = jnp.exp(alpha)
    den = jax.ops.segment_sum(ex, dst, num_segments=N)
    att = ex / (den[dst] + 1e-16)
    out = jax.ops.segment_sum(h[src] * att[..., None], dst, num_segments=N)
    if concat:
        out = out.reshape(N, heads * C)
    else:
        out = out.mean(axis=1)
    return out + b, att


def kernel(rainfall, inflow, edge_index, lstm_W_ih, lstm_W_hh, lstm_b_ih,
           lstm_b_hh, fc_W, fc_b, c1_W, c1_asrc, c1_adst, c1_b, c2_W,
           c2_asrc, c2_adst, c2_b, cell_W_ih, cell_W_hh, cell_b_ih,
           cell_b_hh, ln_g, ln_b, lin_W, lin_b):
    batch = rainfall.shape[0]
    seq_len = rainfall.shape[1]
    N = NUM_NODES
    E = edge_index.shape[1]
    ETOT = batch * E + BN
    NCHUNK = -(-ETOT // (NS * CH))
    ETOTP = NS * NCHUNK * CH

    offs = (jnp.arange(batch, dtype=edge_index.dtype) * N)
    ei = (edge_index[:, None, :] + offs[None, :, None]).reshape(2, -1)
    loops = jnp.arange(BN, dtype=ei.dtype)
    src = jnp.concatenate([ei[0], loops])
    dst = jnp.concatenate([ei[1], loops])
    pad = jnp.full((ETOTP - ETOT,), BN, dtype=jnp.int32)
    srcp = jnp.concatenate([src, pad]).reshape(ETOTP // 128, 128)
    dstp = jnp.concatenate([dst, pad]).reshape(ETOTP // 128, 128)
    z4 = jnp.zeros((BNPAD // NS, 4), jnp.float32)
    z12 = jnp.zeros((BNPAD // NS, 12), jnp.float32)
    z8 = jnp.zeros((BNPAD // NS, 8), jnp.float32)

    edge1 = _make_gat_edge(12, NCHUNK, ETOTP)
    edge2 = _make_gat_edge(8, NCHUNK, ETOTP)

    runoff = _runoff(rainfall, lstm_W_ih, lstm_W_hh, lstm_b_ih, lstm_b_hh,
                     fc_W, fc_b)

    hn = jnp.zeros((batch, H_RT), dtype=jnp.float32)
    cn = jnp.zeros((batch, H_RT), dtype=jnp.float32)
    xn = jnp.zeros((batch, N, 2), dtype=jnp.float32)
    preds, lats, atts = [], [], []
    for t in range(seq_len):
        cr = runoff[:, t, :].at[:, 753].add(inflow[:, t, 0])
        lat = cr[:, :, None]
        x = jnp.concatenate([xn, lat], axis=-1).reshape(BN, 3)
        x, att = _gat_layer_sc(edge1, x, srcp, dstp, z4, z12, c1_W, c1_asrc,
                               c1_adst, c1_b, 3, True, ETOT, ETOTP)
        x = jax.nn.leaky_relu(x, 0.01)
        x, _ = _gat_layer_sc(edge2, x, srcp, dstp, z4, z8, c2_W, c2_asrc,
                             c2_adst, c2_b, 2, False, ETOT, ETOTP)
        x = jax.nn.leaky_relu(x, 0.01)
        x = x.reshape(batch, -1)
        g = x @ cell_W_ih.T + cell_b_ih + hn @ cell_W_hh.T + cell_b_hh
        ig, fg, gg, og = jnp.split(g, 4, axis=1)
        cn = jax.nn.sigmoid(fg) * cn + jax.nn.sigmoid(ig) * jnp.tanh(gg)
        hn = jax.nn.sigmoid(og) * jnp.tanh(cn)
        mu = hn.mean(axis=-1, keepdims=True)
        var = ((hn - mu) ** 2).mean(axis=-1, keepdims=True)
        hn = (hn - mu) / jnp.sqrt(var + 1e-5) * ln_g + ln_b
        xn_flat = jax.nn.softplus(hn @ lin_W.T + lin_b)
        preds.append(xn_flat)
        lats.append(lat)
        atts.append(att)
        xn = xn_flat.reshape(batch, N, 2)
    prediction = jnp.stack(preds, axis=1)
    Lateral = jnp.stack(lats, axis=1)
    Attention = jnp.stack(atts, axis=1)
    return (prediction, Lateral, Attention)


# speedup vs baseline: 44.4615x; 1.0731x over previous
import functools

import jax
import jax.numpy as jnp
from jax import lax
from jax.experimental import pallas as pl
from jax.experimental.pallas import tpu as pltpu
from jax.experimental.pallas import tpu_sc as plsc

NUM_NODES = 10000
B = 4
T = 12
H_RR = 64
H_RT = 256
HEADS = 8

BN = B * NUM_NODES
NC = 2
NS = 16
BNPAD = 40960
CH = 1024
HPC = HEADS // NC


def _runoff_kernel(rain_ref, w_ih_ref, w_hh_ref, b_ih_ref, b_hh_ref,
                   fcw_ref, fcb_ref, out_ref):
    rain = rain_ref[...]
    w_ih = w_ih_ref[...]
    w_hh = w_hh_ref[...]
    b_all = b_ih_ref[...] + b_hh_ref[...]
    h = jnp.zeros((B, H_RR), dtype=jnp.float32)
    c = jnp.zeros((B, H_RR), dtype=jnp.float32)
    outs = []
    for t in range(T):
        g = rain[:, t:t + 1] * w_ih[:, 0][None, :] + b_all
        g = g + jnp.dot(h, w_hh.T, preferred_element_type=jnp.float32)
        ig, fg, gg, og = jnp.split(g, 4, axis=1)
        c = jax.nn.sigmoid(fg) * c + jax.nn.sigmoid(ig) * jnp.tanh(gg)
        h = jax.nn.sigmoid(og) * jnp.tanh(c)
        outs.append(h)
    lstm_out = jnp.concatenate(outs, axis=0)
    r = jnp.dot(lstm_out, fcw_ref[...].T, preferred_element_type=jnp.float32)
    r = r + fcb_ref[...]
    out_ref[...] = jax.nn.leaky_relu(r, 0.01)


def _runoff(rainfall, lstm_W_ih, lstm_W_hh, lstm_b_ih, lstm_b_hh, fc_W, fc_b):
    rain = rainfall[:, :, 0]
    out = pl.pallas_call(
        _runoff_kernel,
        out_shape=jax.ShapeDtypeStruct((T * B, NUM_NODES), jnp.float32),
    )(rain, lstm_W_ih, lstm_W_hh, lstm_b_ih[None, :], lstm_b_hh[None, :],
      fc_W, fc_b[None, :])
    return out.reshape(T, B, NUM_NODES).transpose(1, 0, 2)


def _gat_edge_body(FH, NCHUNK, ETOTP,
                   asrc_hbm, adst_hbm, h_hbm, z4_hbm, zF_hbm,
                   src_hbm, dst_hbm,
                   att_hbm, out_hbm, den_hbm, ex_hbm,
                   den_sh, out_sh,
                   sidx_v, didx_v, didx2_v,
                   asrc_v, adst_v, ex_v, den_v, att_v, h_v, ctr_v,
                   sem):
    c = lax.axis_index("c")
    tid = lax.axis_index("s")
    E_T = NCHUNK * CH
    rows_per_tile = BNPAD // NS
    CF = FH // HPC
    coff = c * BNPAD
    iota = lax.iota(jnp.int32, 16)

    r0 = tid * rows_per_tile
    pltpu.sync_copy(z4_hbm, den_sh.at[pl.ds(r0, rows_per_tile)])
    pltpu.sync_copy(zF_hbm, out_sh.at[pl.ds(r0, rows_per_tile)])
    plsc.subcore_barrier()

    ebase_rows = tid * (E_T // 128)
    nsub = CH // 128

    def load_idx(ch):
        rrow = ebase_rows + ch * nsub
        pltpu.sync_copy(src_hbm.at[pl.ds(rrow, nsub)], sidx_v)
        pltpu.sync_copy(dst_hbm.at[pl.ds(rrow, nsub)], didx_v)
        for jr in range(nsub):
            for jc in range(128 // 16):
                sl = (jr, pl.ds(jc * 16, 16))
                sidx_v[sl] = sidx_v[sl] + coff
                didx2_v[sl] = didx_v[sl] + coff

    def gather_alpha_inputs():
        cps = []
        for j in range(nsub):
            cps.append(pltpu.async_copy(
                asrc_hbm.at[sidx_v.at[j]],
                asrc_v.at[pl.ds(j * 128, 128)], sem))
            cps.append(pltpu.async_copy(
                adst_hbm.at[didx2_v.at[j]],
                adst_v.at[pl.ds(j * 128, 128)], sem))
        for cp in cps:
            cp.wait()

    def compute_ex(k, _):
        j0 = k * 16 + iota
        row = j0 // 4
        col = j0 - row * 4
        a = (plsc.load_gather(asrc_v, [row, col])
             + plsc.load_gather(adst_v, [row, col]))
        a = jnp.where(a > 0, a, 0.2 * a)
        plsc.store_scatter(ex_v, [row, col], jnp.exp(a))
        return 0

    def ex_rows(ch):
        return c * ETOTP + tid * E_T + ch * CH

    def pass1(ch, _):
        load_idx(ch)
        gather_alpha_inputs()
        lax.fori_loop(0, CH * 4 // 16, compute_ex, 0)
        pltpu.sync_copy(ex_v, ex_hbm.at[pl.ds(ex_rows(ch), CH)])
        for j in range(nsub):
            pltpu.sync_copy(ex_v.at[pl.ds(j * 128, 128)],
                            den_sh.at[didx_v.at[j]], add=True)
        return 0

    lax.fori_loop(0, NCHUNK, pass1, 0)
    plsc.subcore_barrier()
    pltpu.sync_copy(den_sh.at[pl.ds(r0, rows_per_tile)],
                    den_hbm.at[pl.ds(coff + r0, rows_per_tile)])
    plsc.subcore_barrier()

    def pass2(ch, _):
        load_idx(ch)
        cps = [pltpu.async_copy(ex_hbm.at[pl.ds(ex_rows(ch), CH)], ex_v, sem)]
        for j in range(nsub):
            cps.append(pltpu.async_copy(
                den_hbm.at[didx2_v.at[j]],
                den_v.at[pl.ds(j * 128, 128)], sem))
            cps.append(pltpu.async_copy(
                h_hbm.at[sidx_v.at[j]],
                h_v.at[pl.ds(j * 128, 128)], sem))
        for cp in cps:
            cp.wait()

        def compute_att(k, _):
            j0 = k * 16 + iota
            row = j0 // 4
            col = j0 - row * 4
            e = plsc.load_gather(ex_v, [row, col])
            d = plsc.load_gather(den_v, [row, col])
            plsc.store_scatter(att_v, [row, col], e / (d + 1e-16))
            return 0

        lax.fori_loop(0, CH * 4 // 16, compute_att, 0)

        def compute_ctr(k, _):
            j0 = k * 16 + iota
            row = j0 // FH
            col = j0 - row * FH
            af = j0 // CF
            arow = af // 4
            acol = af - arow * 4
            av = plsc.load_gather(att_v, [arow, acol])
            hv = plsc.load_gather(h_v, [row, col])
            plsc.store_scatter(ctr_v, [row, col], av * hv)
            return 0

        lax.fori_loop(0, CH * FH // 16, compute_ctr, 0)

        pltpu.sync_copy(att_v, att_hbm.at[pl.ds(ex_rows(ch), CH)])
        for j in range(nsub):
            pltpu.sync_copy(ctr_v.at[pl.ds(j * 128, 128)],
                            out_sh.at[didx_v.at[j]], add=True)
        return 0

    lax.fori_loop(0, NCHUNK, pass2, 0)
    plsc.subcore_barrier()

    pltpu.sync_copy(out_sh.at[pl.ds(r0, rows_per_tile)],
                    out_hbm.at[pl.ds(coff + r0, rows_per_tile)])


def _make_gat_edge(FH, NCHUNK, ETOTP):
    mesh = plsc.VectorSubcoreMesh(core_axis_name="c", subcore_axis_name="s",
                                  num_cores=NC, num_subcores=NS)
    return pl.kernel(
        functools.partial(_gat_edge_body, FH, NCHUNK, ETOTP),
        out_type=[jax.ShapeDtypeStruct((2 * ETOTP, 4), jnp.float32),
                  jax.ShapeDtypeStruct((2 * BNPAD, FH), jnp.float32),
                  jax.ShapeDtypeStruct((2 * BNPAD, 4), jnp.float32),
                  jax.ShapeDtypeStruct((2 * ETOTP, 4), jnp.float32)],
        mesh=mesh,
        compiler_params=pltpu.CompilerParams(use_tc_tiling_on_sc=False,
                                             needs_layout_passes=False),
        scratch_types=[
            pltpu.VMEM_SHARED((BNPAD, 4), jnp.float32),
            pltpu.VMEM_SHARED((BNPAD, FH), jnp.float32),
            pltpu.VMEM((CH // 128, 128), jnp.int32),
            pltpu.VMEM((CH // 128, 128), jnp.int32),
            pltpu.VMEM((CH // 128, 128), jnp.int32),
            pltpu.VMEM((CH, 4), jnp.float32),
            pltpu.VMEM((CH, 4), jnp.float32),
            pltpu.VMEM((CH, 4), jnp.float32),
            pltpu.VMEM((CH, 4), jnp.float32),
            pltpu.VMEM((CH, 4), jnp.float32),
            pltpu.VMEM((CH, FH), jnp.float32),
            pltpu.VMEM((CH, FH), jnp.float32),
            pltpu.SemaphoreType.DMA,
        ],
    )


def _pack_heads(a):
    p = a.reshape(BN, NC, HPC).transpose(1, 0, 2)
    p = jnp.pad(p, ((0, 0), (0, BNPAD - BN), (0, 0)))
    return p.reshape(NC * BNPAD, HPC)


def _pack_h(h, C):
    p = h.reshape(BN, NC, HPC, C).transpose(1, 0, 2, 3)
    p = p.reshape(NC, BN, HPC * C)
    p = jnp.pad(p, ((0, 0), (0, BNPAD - BN), (0, 0)))
    return p.reshape(NC * BNPAD, HPC * C)


def _gat_layer_sc(edge_fn, x, srcp, dstp, z4, zF, W, asrc, adst, b, C,
                  concat, ETOT, ETOTP):
    heads = HEADS
    h = (x @ W.T).reshape(BN, heads, C)
    a_s = (h * asrc).sum(-1)
    a_d = (h * adst).sum(-1)
    att2, out2, _, _ = edge_fn(_pack_heads(a_s), _pack_heads(a_d),
                               _pack_h(h, C), z4, zF, srcp, dstp)
    att = att2.reshape(NC, ETOTP, HPC)[:, :ETOT]
    att = att.transpose(1, 0, 2).reshape(ETOT, heads)
    out = out2.reshape(NC, BNPAD, HPC, C)[:, :BN]
    out = out.transpose(1, 0, 2, 3).reshape(BN, heads, C)
    if concat:
        out = out.reshape(BN, heads * C)
    else:
        out = out.mean(axis=1)
    return out + b, att


def _gat_layer_xla(x, src, dst, W, asrc, adst, b, heads, C, concat, N):
    h = (x @ W.T).reshape(-1, heads, C)
    a_s = (h * asrc).sum(-1)
    a_d = (h * adst).sum(-1)
    alpha = a_s[src] + a_d[dst]
    alpha = jnp.where(alpha > 0, alpha, 0.2 * alpha)
    ex = jnp.exp(alpha)
    den = jax.ops.segment_sum(ex, dst, num_segments=N)
    att = ex / (den[dst] + 1e-16)
    out = jax.ops.segment_sum(h[src] * att[..., None], dst, num_segments=N)
    if concat:
        out = out.reshape(N, heads * C)
    else:
        out = out.mean(axis=1)
    return out + b, att


def kernel(rainfall, inflow, edge_index, lstm_W_ih, lstm_W_hh, lstm_b_ih,
           lstm_b_hh, fc_W, fc_b, c1_W, c1_asrc, c1_adst, c1_b, c2_W,
           c2_asrc, c2_adst, c2_b, cell_W_ih, cell_W_hh, cell_b_ih,
           cell_b_hh, ln_g, ln_b, lin_W, lin_b):
    batch = rainfall.shape[0]
    seq_len = rainfall.shape[1]
    N = NUM_NODES
    E = edge_index.shape[1]
    ETOT = batch * E + BN
    NCHUNK = -(-ETOT // (NS * CH))
    ETOTP = NS * NCHUNK * CH

    offs = (jnp.arange(batch, dtype=edge_index.dtype) * N)
    ei = (edge_index[:, None, :] + offs[None, :, None]).reshape(2, -1)
    loops = jnp.arange(BN, dtype=ei.dtype)
    src = jnp.concatenate([ei[0], loops])
    dst = jnp.concatenate([ei[1], loops])
    pad = jnp.full((ETOTP - ETOT,), BN, dtype=jnp.int32)
    srcp = jnp.concatenate([src, pad]).reshape(ETOTP // 128, 128)
    dstp = jnp.concatenate([dst, pad]).reshape(ETOTP // 128, 128)
    z4 = jnp.zeros((BNPAD // NS, 4), jnp.float32)
    z12 = jnp.zeros((BNPAD // NS, 12), jnp.float32)
    z8 = jnp.zeros((BNPAD // NS, 8), jnp.float32)

    edge1 = _make_gat_edge(12, NCHUNK, ETOTP)
    edge2 = _make_gat_edge(8, NCHUNK, ETOTP)

    runoff = _runoff(rainfall, lstm_W_ih, lstm_W_hh, lstm_b_ih, lstm_b_hh,
                     fc_W, fc_b)

    hn = jnp.zeros((batch, H_RT), dtype=jnp.float32)
    cn = jnp.zeros((batch, H_RT), dtype=jnp.float32)
    xn = jnp.zeros((batch, N, 2), dtype=jnp.float32)
    preds, lats, atts = [], [], []
    for t in range(seq_len):
        cr = runoff[:, t, :].at[:, 753].add(inflow[:, t, 0])
        lat = cr[:, :, None]
        x = jnp.concatenate([xn, lat], axis=-1).reshape(BN, 3)
        x, att = _gat_layer_sc(edge1, x, srcp, dstp, z4, z12, c1_W, c1_asrc,
                               c1_adst, c1_b, 3, True, ETOT, ETOTP)
        x = jax.nn.leaky_relu(x, 0.01)
        x, _ = _gat_layer_sc(edge2, x, srcp, dstp, z4, z8, c2_W, c2_asrc,
                             c2_adst, c2_b, 2, False, ETOT, ETOTP)
        x = jax.nn.leaky_relu(x, 0.01)
        x = x.reshape(batch, -1)
        g = x @ cell_W_ih.T + cell_b_ih + hn @ cell_W_hh.T + cell_b_hh
        ig, fg, gg, og = jnp.split(g, 4, axis=1)
        cn = jax.nn.sigmoid(fg) * cn + jax.nn.sigmoid(ig) * jnp.tanh(gg)
        hn = jax.nn.sigmoid(og) * jnp.tanh(cn)
        mu = hn.mean(axis=-1, keepdims=True)
        var = ((hn - mu) ** 2).mean(axis=-1, keepdims=True)
        hn = (hn - mu) / jnp.sqrt(var + 1e-5) * ln_g + ln_b
        xn_flat = jax.nn.softplus(hn @ lin_W.T + lin_b)
        preds.append(xn_flat)
        lats.append(lat)
        atts.append(att)
        xn = xn_flat.reshape(batch, N, 2)
    prediction = jnp.stack(preds, axis=1)
    Lateral = jnp.stack(lats, axis=1)
    Attention = jnp.stack(atts, axis=1)
    return (prediction, Lateral, Attention)
